# baseline (speedup 1.0000x reference)
import jax
import jax.numpy as jnp
from jax import lax
from jax.experimental import pallas as pl
from jax.experimental.pallas import tpu as pltpu

H = 32
W = 32
HALF = 128
EMBED = 2 * HALF
P = H * W


def _make_pos_kernel(b):
    def _pos_kernel(rows_ref, cols_ref, out_ref, scratch, sem):
        p_idx = lax.broadcasted_iota(jnp.int32, (W, P), 1)
        x_idx = lax.broadcasted_iota(jnp.int32, (W, P), 0)
        sel_c = (p_idx % W == x_idx).astype(jnp.float32)
        sel_r = (p_idx // W == x_idx).astype(jnp.float32)
        cols = cols_ref[0:W, :]
        rows = rows_ref[0:H, :]
        dn = (((0,), (0,)), ((), ()))
        top = lax.dot_general(cols, sel_c, dn,
                              preferred_element_type=jnp.float32,
                              precision=lax.Precision.HIGHEST)
        bot = lax.dot_general(rows, sel_r, dn,
                              preferred_element_type=jnp.float32,
                              precision=lax.Precision.HIGHEST)
        scratch[0:HALF, :] = top
        scratch[HALF:EMBED, :] = bot

        for i in range(b):
            pltpu.make_async_copy(scratch, out_ref.at[i], sem).start()
        for i in range(b):
            pltpu.make_async_copy(scratch, out_ref.at[i], sem).wait()

    return _pos_kernel


def kernel(pixel_values, rows_emb, cols_emb):
    b = pixel_values.shape[0]
    out = pl.pallas_call(
        _make_pos_kernel(b),
        in_specs=[
            pl.BlockSpec(memory_space=pltpu.VMEM),
            pl.BlockSpec(memory_space=pltpu.VMEM),
        ],
        out_specs=pl.BlockSpec(memory_space=pl.ANY),
        out_shape=jax.ShapeDtypeStruct((b, EMBED, P), jnp.float32),
        scratch_shapes=[
            pltpu.VMEM((EMBED, P), jnp.float32),
            pltpu.SemaphoreType.DMA,
        ],
    )(rows_emb, cols_emb)
    return out.reshape(b, EMBED, H, W)
